# baseline (speedup 1.0000x reference)
import jax
import jax.numpy as jnp
from jax.experimental import pallas as pl
from jax.experimental.pallas import tpu as pltpu

SEQ = 4096
B = 4
D = 2048


def _gather_body(idx_ref, src_ref, out_ref, sem):
    copies = [
        pltpu.make_async_copy(src_ref.at[idx_ref[b], b], out_ref.at[b], sem)
        for b in range(B)
    ]
    for c in copies:
        c.start()
    for c in copies:
        c.wait()


_grid_spec = pltpu.PrefetchScalarGridSpec(
    num_scalar_prefetch=1,
    grid=(1,),
    in_specs=[pl.BlockSpec(memory_space=pl.ANY)],
    out_specs=pl.BlockSpec(memory_space=pl.ANY),
    scratch_shapes=[pltpu.SemaphoreType.DMA],
)


def kernel(src, word_pos):
    idx = word_pos.astype(jnp.int32)
    return pl.pallas_call(
        _gather_body,
        grid_spec=_grid_spec,
        out_shape=jax.ShapeDtypeStruct((B, D), jnp.float32),
    )(idx, src)


# speedup vs baseline: 1.3121x; 1.3121x over previous
import jax
import jax.numpy as jnp
from jax.experimental import pallas as pl
from jax.experimental.pallas import tpu as pltpu

SEQ = 4096
B = 4
D = 2048


def _gather_body(idx_ref, src_ref, out_ref, sem):
    copies = [
        pltpu.make_async_copy(src_ref.at[idx_ref[b], b], out_ref.at[b], sem)
        for b in range(B)
    ]
    for c in copies:
        c.start()
    for c in copies:
        c.wait()


def kernel(src, word_pos):
    idx = word_pos.astype(jnp.int32)
    return pl.pallas_call(
        _gather_body,
        in_specs=[
            pl.BlockSpec(memory_space=pltpu.SMEM),
            pl.BlockSpec(memory_space=pl.ANY),
        ],
        out_specs=pl.BlockSpec(memory_space=pltpu.VMEM),
        out_shape=jax.ShapeDtypeStruct((B, D), jnp.float32),
        scratch_shapes=[pltpu.SemaphoreType.DMA],
    )(idx, src)
